# baseline (speedup 1.0000x reference)
import jax
import jax.numpy as jnp
from jax import lax
from jax.experimental import pallas as pl
from jax.experimental.pallas import tpu as pltpu
from jax.experimental.pallas import tpu_sc as plsc

D = 128
B = 16384
NR = 32768
NI = 16384
NM = 16384
NC = 2
NS = 16
L = 16
NPASS = 4
Q = B // (NC * NPASS)
QT = Q // NS
TRASH = Q
ACC_ROWS = Q + 8
REG_CHUNK = NR // NS
SC2_CHUNK = B // NS

f32 = jnp.float32
i32 = jnp.int32


def _leaky(x):
    return jnp.where(x > 0, x, 0.1 * x)


def _imm_pre(x):
    sign = jnp.sign(x)
    mod = jnp.abs(x)
    val = jnp.log2(mod) * sign
    return jnp.where(mod > 2, val, x)


def _compact(idxbuf, tokvals, ctok, cloc, lo, hi, nvec, lane):
    def step(i, n):
        idx = idxbuf[pl.ds(i * L, L)]
        m = (idx >= lo) & (idx < hi)
        mi = m.astype(i32)
        p = n + plsc.cumsum(mi) - 1
        plsc.store_scatter(ctok, [p], tokvals(i), mask=m)
        plsc.store_scatter(cloc, [p >> 7, p & 127], idx - lo, mask=m)
        return n + jnp.sum(mi)

    nmatch = lax.fori_loop(0, nvec, step, jnp.int32(0))
    ceil_ = ((nmatch + 127) >> 7) << 7
    for j in range(8):
        p = nmatch + j * L + lane
        m = p < ceil_
        plsc.store_scatter(ctok, [p], jnp.zeros((L,), i32), mask=m)
        plsc.store_scatter(cloc, [p >> 7, p & 127],
                           jnp.full((L,), TRASH, i32), mask=m)
    return nmatch


NB = 4


def _pipe_gather_scatter(src_hbm, ctok, cloc, bufs, gsems, acc, nmatch, nch):
    for j in range(min(NB, nch)):
        @pl.when(j * 128 < nmatch)
        def _():
            off = pl.multiple_of(j * 128, 128)
            pltpu.async_copy(src_hbm.at[ctok.at[pl.ds(off, 128)]], bufs[j],
                             gsems[j])
    for g in range(nch):
        b = g % NB

        @pl.when(g * 128 < nmatch)
        def _():
            off = pl.multiple_of(g * 128, 128)
            pltpu.make_async_copy(src_hbm.at[ctok.at[pl.ds(off, 128)]],
                                  bufs[b], gsems[b]).wait()
            pltpu.sync_copy(bufs[b], acc.at[cloc.at[g]], add=True)
            if g + NB < nch:
                @pl.when((g + NB) * 128 < nmatch)
                def _():
                    off2 = pl.multiple_of((g + NB) * 128, 128)
                    pltpu.async_copy(src_hbm.at[ctok.at[pl.ds(off2, 128)]],
                                     bufs[b], gsems[b])


def _pipe_gather_out(table, idxbuf, out_hbm, out_base, nch, bufs, gsems):
    for j in range(min(NB, nch)):
        pltpu.async_copy(table.at[idxbuf.at[pl.ds(j * 128, 128)]], bufs[j],
                         gsems[j])
    for k in range(nch):
        b = k % NB
        pltpu.make_async_copy(table.at[idxbuf.at[pl.ds(k * 128, 128)]],
                              bufs[b], gsems[b]).wait()
        pltpu.sync_copy(bufs[b], out_hbm.at[pl.ds(out_base + k * 128, 128)])
        if k + NB < nch:
            pltpu.async_copy(table.at[idxbuf.at[pl.ds((k + NB) * 128, 128)]],
                             bufs[b], gsems[b])


def _sc1_body(table, mnemic, reg_tok, reg_idx, mem_tok, z128, o128,
              mnem_g, memcat, sreg, cnt,
              idxbuf, tokbuf, ctok, cloc, buf0, buf1, buf2, buf3,
              gsem0, gsem1, gsem2, gsem3, ssemA, acc):
    c = lax.axis_index("c")
    s = lax.axis_index("s")
    gid = c * NS + s
    lane = jnp.arange(L, dtype=i32)
    bufs = (buf0, buf1, buf2, buf3)
    gsems = (gsem0, gsem1, gsem2, gsem3)

    mbase = pl.multiple_of(gid * 512, 512)
    pltpu.sync_copy(mnemic.at[pl.ds(mbase, 512)], idxbuf.at[pl.ds(0, 512)])
    _pipe_gather_out(table, idxbuf, mnem_g, mbase, 4, bufs, gsems)

    tbase = pl.multiple_of(gid * 1536, 512)
    pltpu.sync_copy(mem_tok.at[pl.ds(tbase, 1536)], idxbuf.at[pl.ds(0, 1536)])
    _pipe_gather_out(table, idxbuf, memcat, tbase, 12, bufs, gsems)

    base = pl.multiple_of(s * REG_CHUNK, REG_CHUNK)
    pltpu.sync_copy(reg_idx.at[pl.ds(base, REG_CHUNK)], idxbuf)
    pltpu.sync_copy(reg_tok.at[pl.ds(base, REG_CHUNK)], tokbuf)
    srow = pl.multiple_of(s * QT, QT)

    for q in range(NPASS):
        lo = c * (NPASS * Q) + q * Q
        obase = pl.multiple_of(lo + s * QT, QT)

        pltpu.sync_copy(z128, buf0)
        pltpu.async_copy(buf0, acc.at[pl.ds(srow, QT)], ssemA)
        nmatch = _compact(idxbuf, lambda i: tokbuf[pl.ds(i * L, L)],
                          ctok, cloc, lo, lo + Q, REG_CHUNK // L, lane)
        pltpu.make_async_copy(buf0, acc.at[pl.ds(srow, QT)], ssemA).wait()
        plsc.subcore_barrier()

        _pipe_gather_scatter(table, ctok, cloc, bufs, gsems, acc,
                             nmatch, REG_CHUNK // 128)
        plsc.subcore_barrier()
        pltpu.sync_copy(acc.at[pl.ds(srow, QT)], buf0)
        pltpu.async_copy(buf0, sreg.at[pl.ds(obase, QT)], ssemA)

        pltpu.sync_copy(z128, buf1)
        pltpu.sync_copy(o128, buf2)
        pltpu.make_async_copy(buf0, sreg.at[pl.ds(obase, QT)], ssemA).wait()
        pltpu.sync_copy(buf1, acc.at[pl.ds(srow, QT)])
        plsc.subcore_barrier()

        for g in range(REG_CHUNK // 128):
            @pl.when(g * 128 < nmatch)
            def _():
                pltpu.async_copy(buf2, acc.at[cloc.at[g]], ssemA, add=True)
        for g in range(REG_CHUNK // 128):
            @pl.when(g * 128 < nmatch)
            def _():
                pltpu.make_async_copy(buf2, acc.at[cloc.at[0]], ssemA).wait()

        plsc.subcore_barrier()
        pltpu.sync_copy(acc.at[pl.ds(srow, QT)], buf0)
        pltpu.sync_copy(buf0, cnt.at[pl.ds(obase, QT)])


_sc1 = pl.kernel(
    _sc1_body,
    out_type=(
        jax.ShapeDtypeStruct((B, D), f32),
        jax.ShapeDtypeStruct((3 * NM, D), f32),
        jax.ShapeDtypeStruct((B, D), f32),
        jax.ShapeDtypeStruct((B, D), f32),
    ),
    mesh=plsc.VectorSubcoreMesh(core_axis_name="c", subcore_axis_name="s",
                                num_cores=NC, num_subcores=NS),
    scratch_types=(
        pltpu.VMEM((REG_CHUNK,), i32),
        pltpu.VMEM((REG_CHUNK,), i32),
        pltpu.VMEM((REG_CHUNK,), i32),
        pltpu.VMEM((REG_CHUNK // 128, 128), i32),
        pltpu.VMEM((128, D), f32),
        pltpu.VMEM((128, D), f32),
        pltpu.VMEM((128, D), f32),
        pltpu.VMEM((128, D), f32),
        pltpu.SemaphoreType.DMA,
        pltpu.SemaphoreType.DMA,
        pltpu.SemaphoreType.DMA,
        pltpu.SemaphoreType.DMA,
        pltpu.SemaphoreType.DMA,
        pltpu.VMEM_SHARED((ACC_ROWS, D), f32),
    ),
    compiler_params=pltpu.CompilerParams(needs_layout_passes=False),
)


def _sc2_body(rm, ops1, ops2, idx_imm, idx_mem, m_out,
              idxbuf, ctok, cloc, buf0, buf1, buf2, buf3,
              gsem0, gsem1, gsem2, gsem3, ssemA, acc):
    c = lax.axis_index("c")
    s = lax.axis_index("s")
    lane = jnp.arange(L, dtype=i32)
    srow = pl.multiple_of(s * QT, QT)
    base = pl.multiple_of(s * SC2_CHUNK, SC2_CHUNK)
    bufs = (buf0, buf1, buf2, buf3)
    gsems = (gsem0, gsem1, gsem2, gsem3)

    for q in range(NPASS):
        lo = c * (NPASS * Q) + q * Q
        obase = pl.multiple_of(lo + s * QT, QT)
        pltpu.sync_copy(rm.at[pl.ds(obase, QT)], buf0)
        pltpu.sync_copy(buf0, acc.at[pl.ds(srow, QT)])
        plsc.subcore_barrier()

        for idx_hbm, src_hbm in ((idx_imm, ops1), (idx_mem, ops2)):
            pltpu.sync_copy(idx_hbm.at[pl.ds(base, SC2_CHUNK)], idxbuf)
            nmatch = _compact(idxbuf, lambda i: base + i * L + lane,
                              ctok, cloc, lo, lo + Q, SC2_CHUNK // L, lane)
            _pipe_gather_scatter(src_hbm, ctok, cloc, bufs, gsems,
                                 acc, nmatch, SC2_CHUNK // 128)

        plsc.subcore_barrier()
        pltpu.sync_copy(acc.at[pl.ds(srow, QT)], buf0)
        pltpu.sync_copy(buf0, m_out.at[pl.ds(obase, QT)])


_sc2 = pl.kernel(
    _sc2_body,
    out_type=jax.ShapeDtypeStruct((B, D), f32),
    mesh=plsc.VectorSubcoreMesh(core_axis_name="c", subcore_axis_name="s",
                                num_cores=NC, num_subcores=NS),
    scratch_types=(
        pltpu.VMEM((SC2_CHUNK,), i32),
        pltpu.VMEM((SC2_CHUNK,), i32),
        pltpu.VMEM((SC2_CHUNK // 128, 128), i32),
        pltpu.VMEM((128, D), f32),
        pltpu.VMEM((128, D), f32),
        pltpu.VMEM((128, D), f32),
        pltpu.VMEM((128, D), f32),
        pltpu.SemaphoreType.DMA,
        pltpu.SemaphoreType.DMA,
        pltpu.SemaphoreType.DMA,
        pltpu.SemaphoreType.DMA,
        pltpu.SemaphoreType.DMA,
        pltpu.VMEM_SHARED((ACC_ROWS, D), f32),
    ),
    compiler_params=pltpu.CompilerParams(needs_layout_passes=False),
)


RB = 1024


def _tcb_body(imm_ref, disp_ref, memcat_ref, sreg_ref, cnt_ref, mnem_ref,
              iW1, ib1, iW2, ib2, mW1, mb1, mW2, mb2,
              oW0, ob0, oW1, ob1, oW2, ob2,
              ops1_ref, ops2_ref, r_ref):
    w1i = iW1[...]
    b1i = ib1[...][None, :]
    w2i = iW2[...]
    b2i = ib2[...][None, :]

    x = _imm_pre(imm_ref[...])
    h = _leaky(x * w1i + b1i)
    imm_e = _leaky(jnp.dot(h, w2i, preferred_element_type=f32) + b2i)
    ops1_ref[...] = jnp.dot(imm_e, oW1[...], preferred_element_type=f32) \
        + ob1[...][None, :]

    dx = _imm_pre(disp_ref[...])
    hd = _leaky(dx * w1i + b1i)
    disp_e = _leaky(jnp.dot(hd, w2i, preferred_element_type=f32) + b2i)

    w1m = mW1[...]
    m1 = (jnp.dot(memcat_ref[...], w1m[:3 * D], preferred_element_type=f32)
          + jnp.dot(disp_e, w1m[3 * D:], preferred_element_type=f32)
          + mb1[...][None, :])
    mem_e = _leaky(jnp.dot(_leaky(m1), mW2[...], preferred_element_type=f32)
                   + mb2[...][None, :])
    ops2_ref[...] = jnp.dot(mem_e, oW2[...], preferred_element_type=f32) \
        + ob2[...][None, :]

    r_ref[...] = (mnem_ref[...]
                  + jnp.dot(sreg_ref[...], oW0[...], preferred_element_type=f32)
                  + cnt_ref[:, 0:1] * ob0[...][None, :])


def _row_spec(cols):
    return pl.BlockSpec((RB, cols), lambda i: (i, 0))


def _full2(shape):
    return pl.BlockSpec(shape, lambda i: (0, 0))


def _full1(n):
    return pl.BlockSpec((n,), lambda i: (0,))


_tcb = pl.pallas_call(
    _tcb_body,
    grid=(B // RB,),
    in_specs=[
        _row_spec(1), _row_spec(1), _row_spec(3 * D), _row_spec(D),
        _row_spec(D), _row_spec(D),
        _full2((1, D)), _full1(D), _full2((D, D)), _full1(D),
        _full2((4 * D, D)), _full1(D), _full2((D, D)), _full1(D),
        _full2((D, D)), _full1(D), _full2((D, D)), _full1(D),
        _full2((D, D)), _full1(D),
    ],
    out_specs=[_row_spec(D)] * 3,
    out_shape=[jax.ShapeDtypeStruct((B, D), f32)] * 3,
)


def _tcd_body(m_ref, aggW_ref, aggb_ref, out_ref):
    out_ref[...] = _leaky(jnp.dot(_leaky(m_ref[...]), aggW_ref[...],
                                  preferred_element_type=f32)
                          + aggb_ref[...][None, :])


_tcd = pl.pallas_call(
    _tcd_body,
    grid=(B // RB,),
    in_specs=[_row_spec(D), _full2((D, D)), _full1(D)],
    out_specs=_row_spec(D),
    out_shape=jax.ShapeDtypeStruct((B, D), f32),
)


def kernel(mnemic, reg_tokens, imm_vals, mem_tokens, mem_disp,
           ins_idx_reg, ins_idx_imm, ins_idx_mem,
           table, imm_W1, imm_b1, imm_W2, imm_b2,
           mem_W1, mem_b1, mem_W2, mem_b2,
           opW0, opb0, opW1, opb1, opW2, opb2, aggW, aggb):
    mnemic = mnemic.astype(i32)
    reg_tokens = reg_tokens.astype(i32)
    mem_flat = mem_tokens.astype(i32).reshape(-1)
    iir = ins_idx_reg.astype(i32)
    iii = ins_idx_imm.astype(i32)
    iim = ins_idx_mem.astype(i32)
    z128 = jnp.zeros((128, D), f32)
    o128 = jnp.ones((128, D), f32)

    mnem_g, memcat, sreg, cnt = _sc1(table, mnemic, reg_tokens, iir,
                                     mem_flat, z128, o128)
    ops1, ops2, r = _tcb(imm_vals, mem_disp.reshape(NM, 1),
                         memcat.reshape(NM, 3 * D), sreg, cnt, mnem_g,
                         imm_W1, imm_b1, imm_W2, imm_b2,
                         mem_W1, mem_b1, mem_W2, mem_b2,
                         opW0, opb0, opW1, opb1, opW2, opb2)
    m = _sc2(r, ops1, ops2, iii, iim)
    return _tcd(m, aggW, aggb)


# speedup vs baseline: 1.6628x; 1.6628x over previous
import jax
import jax.numpy as jnp
from jax import lax
from jax.experimental import pallas as pl
from jax.experimental.pallas import tpu as pltpu
from jax.experimental.pallas import tpu_sc as plsc

D = 128
B = 16384
NR = 32768
NI = 16384
NM = 16384
NC = 2
NS = 16
L = 16
NPASS = 2
Q = B // (NC * NPASS)
QT = Q // NS
TRASH = Q
ACC_ROWS = Q + 8
REG_CHUNK = NR // NS
SC2_CHUNK = B // NS

f32 = jnp.float32
i32 = jnp.int32


def _leaky(x):
    return jnp.where(x > 0, x, 0.1 * x)


def _imm_pre(x):
    sign = jnp.sign(x)
    mod = jnp.abs(x)
    val = jnp.log2(mod) * sign
    return jnp.where(mod > 2, val, x)


_BCAST_DNUMS = lax.GatherDimensionNumbers(
    offset_dims=(), collapsed_slice_dims=(0,), start_index_map=(0,))


def _bcast_last(v):
    return lax.gather(v, jnp.full((L, 1), L - 1, i32), _BCAST_DNUMS, (1,),
                      mode=lax.GatherScatterMode.PROMISE_IN_BOUNDS)


def _compact(idxbuf, tokvals, ctok, cloc, lo, hi, nvec, lane):
    def step(i, nv):
        idx = idxbuf[pl.ds(i * L, L)]
        m = (idx >= lo) & (idx < hi)
        mi = m.astype(i32)
        cs = plsc.cumsum(mi)
        p = nv + cs - 1
        plsc.store_scatter(ctok, [p], tokvals(i), mask=m)
        plsc.store_scatter(cloc, [p >> 7, p & 127], idx - lo, mask=m)
        return nv + _bcast_last(cs)

    nv = lax.fori_loop(0, nvec, step, jnp.zeros((L,), i32))
    nmatch = jnp.max(nv)
    ceil_ = ((nmatch + 127) >> 7) << 7
    for j in range(8):
        p = nmatch + j * L + lane
        m = p < ceil_
        plsc.store_scatter(ctok, [p], jnp.zeros((L,), i32), mask=m)
        plsc.store_scatter(cloc, [p >> 7, p & 127],
                           jnp.full((L,), TRASH, i32), mask=m)
    return nmatch


NB = 3


def _pipe_gather_scatter(src_hbm, ctok, cloc, bufs, gsems, acc, nmatch, nch):
    for j in range(min(NB, nch)):
        @pl.when(j * 128 < nmatch)
        def _():
            off = pl.multiple_of(j * 128, 128)
            pltpu.async_copy(src_hbm.at[ctok.at[pl.ds(off, 128)]], bufs[j],
                             gsems[j])
    for g in range(nch):
        b = g % NB

        @pl.when(g * 128 < nmatch)
        def _():
            off = pl.multiple_of(g * 128, 128)
            pltpu.make_async_copy(src_hbm.at[ctok.at[pl.ds(off, 128)]],
                                  bufs[b], gsems[b]).wait()
            pltpu.sync_copy(bufs[b], acc.at[cloc.at[g]], add=True)
            if g + NB < nch:
                @pl.when((g + NB) * 128 < nmatch)
                def _():
                    off2 = pl.multiple_of((g + NB) * 128, 128)
                    pltpu.async_copy(src_hbm.at[ctok.at[pl.ds(off2, 128)]],
                                     bufs[b], gsems[b])


def _pipe_gather_out(table, idxbuf, out_hbm, out_base, nch, bufs, gsems):
    for j in range(min(NB, nch)):
        pltpu.async_copy(table.at[idxbuf.at[pl.ds(j * 128, 128)]], bufs[j],
                         gsems[j])
    for k in range(nch):
        b = k % NB
        pltpu.make_async_copy(table.at[idxbuf.at[pl.ds(k * 128, 128)]],
                              bufs[b], gsems[b]).wait()
        pltpu.sync_copy(bufs[b], out_hbm.at[pl.ds(out_base + k * 128, 128)])
        if k + NB < nch:
            pltpu.async_copy(table.at[idxbuf.at[pl.ds((k + NB) * 128, 128)]],
                             bufs[b], gsems[b])


def _sc1_body(table, mnemic, reg_tok, reg_idx, mem_tok, z128, o128,
              mnem_g, memcat, sreg, cnt,
              idxbuf, tokbuf, ctok, cloc, buf0, buf1, buf2,
              gsem0, gsem1, gsem2, ssemA, acc):
    c = lax.axis_index("c")
    s = lax.axis_index("s")
    gid = c * NS + s
    lane = jnp.arange(L, dtype=i32)
    bufs = (buf0, buf1, buf2)
    gsems = (gsem0, gsem1, gsem2)

    mbase = pl.multiple_of(gid * 512, 512)
    pltpu.sync_copy(mnemic.at[pl.ds(mbase, 512)], idxbuf.at[pl.ds(0, 512)])
    _pipe_gather_out(table, idxbuf, mnem_g, mbase, 4, bufs, gsems)

    tbase = pl.multiple_of(gid * 1536, 512)
    pltpu.sync_copy(mem_tok.at[pl.ds(tbase, 1536)], idxbuf.at[pl.ds(0, 1536)])
    _pipe_gather_out(table, idxbuf, memcat, tbase, 12, bufs, gsems)

    base = pl.multiple_of(s * REG_CHUNK, REG_CHUNK)
    pltpu.sync_copy(reg_idx.at[pl.ds(base, REG_CHUNK)], idxbuf)
    pltpu.sync_copy(reg_tok.at[pl.ds(base, REG_CHUNK)], tokbuf)
    srow = pl.multiple_of(s * QT, QT)

    for q in range(NPASS):
        lo = c * (NPASS * Q) + q * Q
        obase = pl.multiple_of(lo + s * QT, QT)

        pltpu.sync_copy(z128, buf0)
        pltpu.async_copy(buf0, acc.at[pl.ds(srow, 128)], ssemA)
        pltpu.async_copy(buf0, acc.at[pl.ds(srow + 128, 128)], ssemA)
        nmatch = _compact(idxbuf, lambda i: tokbuf[pl.ds(i * L, L)],
                          ctok, cloc, lo, lo + Q, REG_CHUNK // L, lane)
        pltpu.make_async_copy(buf0, acc.at[pl.ds(srow, 128)], ssemA).wait()
        pltpu.make_async_copy(buf0, acc.at[pl.ds(srow, 128)], ssemA).wait()
        plsc.subcore_barrier()

        _pipe_gather_scatter(table, ctok, cloc, bufs, gsems, acc,
                             nmatch, REG_CHUNK // 128)
        plsc.subcore_barrier()
        pltpu.sync_copy(acc.at[pl.ds(srow, 128)], buf0)
        pltpu.async_copy(buf0, sreg.at[pl.ds(obase, 128)], ssemA)
        pltpu.sync_copy(acc.at[pl.ds(srow + 128, 128)], buf1)
        pltpu.async_copy(buf1, sreg.at[pl.ds(obase + 128, 128)], ssemA)

        pltpu.sync_copy(z128, buf2)
        pltpu.make_async_copy(buf0, sreg.at[pl.ds(obase, 128)], ssemA).wait()
        pltpu.make_async_copy(buf0, sreg.at[pl.ds(obase, 128)], ssemA).wait()
        pltpu.sync_copy(buf2, acc.at[pl.ds(srow, 128)])
        pltpu.sync_copy(buf2, acc.at[pl.ds(srow + 128, 128)])
        pltpu.sync_copy(o128, buf2)
        plsc.subcore_barrier()

        for g in range(REG_CHUNK // 128):
            @pl.when(g * 128 < nmatch)
            def _():
                pltpu.async_copy(buf2, acc.at[cloc.at[g]], ssemA, add=True)
        for g in range(REG_CHUNK // 128):
            @pl.when(g * 128 < nmatch)
            def _():
                pltpu.make_async_copy(buf2, acc.at[cloc.at[0]], ssemA).wait()

        plsc.subcore_barrier()
        pltpu.sync_copy(acc.at[pl.ds(srow, 128)], buf0)
        pltpu.async_copy(buf0, cnt.at[pl.ds(obase, 128)], ssemA)
        pltpu.sync_copy(acc.at[pl.ds(srow + 128, 128)], buf1)
        pltpu.sync_copy(buf1, cnt.at[pl.ds(obase + 128, 128)])
        pltpu.make_async_copy(buf0, cnt.at[pl.ds(obase, 128)], ssemA).wait()


_sc1 = pl.kernel(
    _sc1_body,
    out_type=(
        jax.ShapeDtypeStruct((B, D), f32),
        jax.ShapeDtypeStruct((3 * NM, D), f32),
        jax.ShapeDtypeStruct((B, D), f32),
        jax.ShapeDtypeStruct((B, D), f32),
    ),
    mesh=plsc.VectorSubcoreMesh(core_axis_name="c", subcore_axis_name="s",
                                num_cores=NC, num_subcores=NS),
    scratch_types=(
        pltpu.VMEM((REG_CHUNK,), i32),
        pltpu.VMEM((REG_CHUNK,), i32),
        pltpu.VMEM((REG_CHUNK,), i32),
        pltpu.VMEM((REG_CHUNK // 128, 128), i32),
        pltpu.VMEM((128, D), f32),
        pltpu.VMEM((128, D), f32),
        pltpu.VMEM((128, D), f32),
        pltpu.SemaphoreType.DMA,
        pltpu.SemaphoreType.DMA,
        pltpu.SemaphoreType.DMA,
        pltpu.SemaphoreType.DMA,
        pltpu.VMEM_SHARED((ACC_ROWS, D), f32),
    ),
    compiler_params=pltpu.CompilerParams(needs_layout_passes=False),
)


def _sc2_body(rm, ops1, ops2, idx_imm, idx_mem, m_out,
              idxbuf, ctok, cloc, buf0, buf1, buf2,
              gsem0, gsem1, gsem2, ssemA, acc):
    c = lax.axis_index("c")
    s = lax.axis_index("s")
    lane = jnp.arange(L, dtype=i32)
    srow = pl.multiple_of(s * QT, QT)
    base = pl.multiple_of(s * SC2_CHUNK, SC2_CHUNK)
    bufs = (buf0, buf1, buf2)
    gsems = (gsem0, gsem1, gsem2)

    for q in range(NPASS):
        lo = c * (NPASS * Q) + q * Q
        obase = pl.multiple_of(lo + s * QT, QT)
        pltpu.sync_copy(rm.at[pl.ds(obase, 128)], buf0)
        pltpu.sync_copy(rm.at[pl.ds(obase + 128, 128)], buf1)
        pltpu.sync_copy(buf0, acc.at[pl.ds(srow, 128)])
        pltpu.sync_copy(buf1, acc.at[pl.ds(srow + 128, 128)])
        plsc.subcore_barrier()

        for idx_hbm, src_hbm in ((idx_imm, ops1), (idx_mem, ops2)):
            pltpu.sync_copy(idx_hbm.at[pl.ds(base, SC2_CHUNK)], idxbuf)
            nmatch = _compact(idxbuf, lambda i: base + i * L + lane,
                              ctok, cloc, lo, lo + Q, SC2_CHUNK // L, lane)
            _pipe_gather_scatter(src_hbm, ctok, cloc, bufs, gsems,
                                 acc, nmatch, SC2_CHUNK // 128)

        plsc.subcore_barrier()
        pltpu.sync_copy(acc.at[pl.ds(srow, 128)], buf0)
        pltpu.async_copy(buf0, m_out.at[pl.ds(obase, 128)], ssemA)
        pltpu.sync_copy(acc.at[pl.ds(srow + 128, 128)], buf1)
        pltpu.sync_copy(buf1, m_out.at[pl.ds(obase + 128, 128)])
        pltpu.make_async_copy(buf0, m_out.at[pl.ds(obase, 128)], ssemA).wait()


_sc2 = pl.kernel(
    _sc2_body,
    out_type=jax.ShapeDtypeStruct((B, D), f32),
    mesh=plsc.VectorSubcoreMesh(core_axis_name="c", subcore_axis_name="s",
                                num_cores=NC, num_subcores=NS),
    scratch_types=(
        pltpu.VMEM((SC2_CHUNK,), i32),
        pltpu.VMEM((SC2_CHUNK,), i32),
        pltpu.VMEM((SC2_CHUNK // 128, 128), i32),
        pltpu.VMEM((128, D), f32),
        pltpu.VMEM((128, D), f32),
        pltpu.VMEM((128, D), f32),
        pltpu.SemaphoreType.DMA,
        pltpu.SemaphoreType.DMA,
        pltpu.SemaphoreType.DMA,
        pltpu.SemaphoreType.DMA,
        pltpu.VMEM_SHARED((ACC_ROWS, D), f32),
    ),
    compiler_params=pltpu.CompilerParams(needs_layout_passes=False),
)


RB = 1024


def _tcb_body(imm_ref, disp_ref, memcat_ref, sreg_ref, cnt_ref, mnem_ref,
              iW1, ib1, iW2, ib2, mW1, mb1, mW2, mb2,
              oW0, ob0, oW1, ob1, oW2, ob2,
              ops1_ref, ops2_ref, r_ref):
    w1i = iW1[...]
    b1i = ib1[...][None, :]
    w2i = iW2[...]
    b2i = ib2[...][None, :]

    x = _imm_pre(imm_ref[...])
    h = _leaky(x * w1i + b1i)
    imm_e = _leaky(jnp.dot(h, w2i, preferred_element_type=f32) + b2i)
    ops1_ref[...] = jnp.dot(imm_e, oW1[...], preferred_element_type=f32) \
        + ob1[...][None, :]

    dx = _imm_pre(disp_ref[...])
    hd = _leaky(dx * w1i + b1i)
    disp_e = _leaky(jnp.dot(hd, w2i, preferred_element_type=f32) + b2i)

    w1m = mW1[...]
    m1 = (jnp.dot(memcat_ref[...], w1m[:3 * D], preferred_element_type=f32)
          + jnp.dot(disp_e, w1m[3 * D:], preferred_element_type=f32)
          + mb1[...][None, :])
    mem_e = _leaky(jnp.dot(_leaky(m1), mW2[...], preferred_element_type=f32)
                   + mb2[...][None, :])
    ops2_ref[...] = jnp.dot(mem_e, oW2[...], preferred_element_type=f32) \
        + ob2[...][None, :]

    r_ref[...] = (mnem_ref[...]
                  + jnp.dot(sreg_ref[...], oW0[...], preferred_element_type=f32)
                  + cnt_ref[:, 0:1] * ob0[...][None, :])


def _row_spec(cols):
    return pl.BlockSpec((RB, cols), lambda i: (i, 0))


def _full2(shape):
    return pl.BlockSpec(shape, lambda i: (0, 0))


def _full1(n):
    return pl.BlockSpec((n,), lambda i: (0,))


_tcb = pl.pallas_call(
    _tcb_body,
    grid=(B // RB,),
    in_specs=[
        _row_spec(1), _row_spec(1), _row_spec(3 * D), _row_spec(D),
        _row_spec(D), _row_spec(D),
        _full2((1, D)), _full1(D), _full2((D, D)), _full1(D),
        _full2((4 * D, D)), _full1(D), _full2((D, D)), _full1(D),
        _full2((D, D)), _full1(D), _full2((D, D)), _full1(D),
        _full2((D, D)), _full1(D),
    ],
    out_specs=[_row_spec(D)] * 3,
    out_shape=[jax.ShapeDtypeStruct((B, D), f32)] * 3,
)


def _tcd_body(m_ref, aggW_ref, aggb_ref, out_ref):
    out_ref[...] = _leaky(jnp.dot(_leaky(m_ref[...]), aggW_ref[...],
                                  preferred_element_type=f32)
                          + aggb_ref[...][None, :])


_tcd = pl.pallas_call(
    _tcd_body,
    grid=(B // RB,),
    in_specs=[_row_spec(D), _full2((D, D)), _full1(D)],
    out_specs=_row_spec(D),
    out_shape=jax.ShapeDtypeStruct((B, D), f32),
)


def kernel(mnemic, reg_tokens, imm_vals, mem_tokens, mem_disp,
           ins_idx_reg, ins_idx_imm, ins_idx_mem,
           table, imm_W1, imm_b1, imm_W2, imm_b2,
           mem_W1, mem_b1, mem_W2, mem_b2,
           opW0, opb0, opW1, opb1, opW2, opb2, aggW, aggb):
    mnemic = mnemic.astype(i32)
    reg_tokens = reg_tokens.astype(i32)
    mem_flat = mem_tokens.astype(i32).reshape(-1)
    iir = ins_idx_reg.astype(i32)
    iii = ins_idx_imm.astype(i32)
    iim = ins_idx_mem.astype(i32)
    z128 = jnp.zeros((128, D), f32)
    o128 = jnp.ones((128, D), f32)

    mnem_g, memcat, sreg, cnt = _sc1(table, mnemic, reg_tokens, iir,
                                     mem_flat, z128, o128)
    ops1, ops2, r = _tcb(imm_vals, mem_disp.reshape(NM, 1),
                         memcat.reshape(NM, 3 * D), sreg, cnt, mnem_g,
                         imm_W1, imm_b1, imm_W2, imm_b2,
                         mem_W1, mem_b1, mem_W2, mem_b2,
                         opW0, opb0, opW1, opb1, opW2, opb2)
    m = _sc2(r, ops1, ops2, iii, iim)
    return _tcd(m, aggW, aggb)


# speedup vs baseline: 1.7462x; 1.0502x over previous
import jax
import jax.numpy as jnp
from jax import lax
from jax.experimental import pallas as pl
from jax.experimental.pallas import tpu as pltpu
from jax.experimental.pallas import tpu_sc as plsc

D = 128
B = 16384
NR = 32768
NI = 16384
NM = 16384
NC = 2
NS = 16
L = 16
NPASS = 2
Q = B // (NC * NPASS)
QT = Q // NS
TRASH = Q
ACC_ROWS = Q + 8
REG_CHUNK = NR // NS
SC2_CHUNK = B // NS

f32 = jnp.float32
i32 = jnp.int32


def _leaky(x):
    return jnp.where(x > 0, x, 0.1 * x)


def _imm_pre(x):
    sign = jnp.sign(x)
    mod = jnp.abs(x)
    val = jnp.log2(mod) * sign
    return jnp.where(mod > 2, val, x)


_BCAST_DNUMS = lax.GatherDimensionNumbers(
    offset_dims=(), collapsed_slice_dims=(0,), start_index_map=(0,))


def _bcast_last(v):
    return lax.gather(v, jnp.full((L, 1), L - 1, i32), _BCAST_DNUMS, (1,),
                      mode=lax.GatherScatterMode.PROMISE_IN_BOUNDS)


def _compact(idxbuf, tokvals, ctok, cloc, lo, hi, nvec, lane):
    def step(i, nv):
        idx = idxbuf[pl.ds(i * L, L)]
        m = (idx >= lo) & (idx < hi)
        mi = m.astype(i32)
        cs = plsc.cumsum(mi)
        p = nv + cs - 1
        plsc.store_scatter(ctok, [p], tokvals(i), mask=m)
        plsc.store_scatter(cloc, [p >> 7, p & 127], idx - lo, mask=m)
        return nv + _bcast_last(cs)

    nv = lax.fori_loop(0, nvec, step, jnp.zeros((L,), i32))
    nmatch = jnp.max(nv)
    ceil_ = ((nmatch + 127) >> 7) << 7
    for j in range(8):
        p = nmatch + j * L + lane
        m = p < ceil_
        plsc.store_scatter(ctok, [p], jnp.zeros((L,), i32), mask=m)
        plsc.store_scatter(cloc, [p >> 7, p & 127],
                           jnp.full((L,), TRASH, i32), mask=m)
    return nmatch


NB = 3


def _pipe_gather_scatter(src_hbm, ctok, cloc, bufs, gsems, acc, nmatch, nch):
    for j in range(min(NB, nch)):
        @pl.when(j * 128 < nmatch)
        def _():
            off = pl.multiple_of(j * 128, 128)
            pltpu.async_copy(src_hbm.at[ctok.at[pl.ds(off, 128)]], bufs[j],
                             gsems[j])
    for g in range(nch):
        b = g % NB

        @pl.when(g * 128 < nmatch)
        def _():
            off = pl.multiple_of(g * 128, 128)
            pltpu.make_async_copy(src_hbm.at[ctok.at[pl.ds(off, 128)]],
                                  bufs[b], gsems[b]).wait()
            pltpu.sync_copy(bufs[b], acc.at[cloc.at[g]], add=True)
            if g + NB < nch:
                @pl.when((g + NB) * 128 < nmatch)
                def _():
                    off2 = pl.multiple_of((g + NB) * 128, 128)
                    pltpu.async_copy(src_hbm.at[ctok.at[pl.ds(off2, 128)]],
                                     bufs[b], gsems[b])


def _pipe_gather_out(table, idxbuf, out_hbm, out_base, nch, bufs, gsems):
    for j in range(min(NB, nch)):
        pltpu.async_copy(table.at[idxbuf.at[pl.ds(j * 128, 128)]], bufs[j],
                         gsems[j])
    for k in range(nch):
        b = k % NB
        pltpu.make_async_copy(table.at[idxbuf.at[pl.ds(k * 128, 128)]],
                              bufs[b], gsems[b]).wait()
        pltpu.sync_copy(bufs[b], out_hbm.at[pl.ds(out_base + k * 128, 128)])
        if k + NB < nch:
            pltpu.async_copy(table.at[idxbuf.at[pl.ds((k + NB) * 128, 128)]],
                             bufs[b], gsems[b])


def _sc1_body(table, mnemic, reg_tok, reg_idx, mem_tok, z128, o128,
              mnem_g, memcat0, memcat1, memcat2, sreg, cnt,
              idxbuf, tokbuf, ctok, cloc, buf0, buf1, buf2,
              gsem0, gsem1, gsem2, ssemA, acc):
    c = lax.axis_index("c")
    s = lax.axis_index("s")
    gid = c * NS + s
    lane = jnp.arange(L, dtype=i32)
    bufs = (buf0, buf1, buf2)
    gsems = (gsem0, gsem1, gsem2)

    mbase = pl.multiple_of(gid * 512, 512)
    pltpu.sync_copy(mnemic.at[pl.ds(mbase, 512)], idxbuf.at[pl.ds(0, 512)])
    _pipe_gather_out(table, idxbuf, mnem_g, mbase, 4, bufs, gsems)

    for k3, mc in enumerate((memcat0, memcat1, memcat2)):
        kbase = pl.multiple_of(k3 * NM + gid * 512, 512)
        pltpu.sync_copy(mem_tok.at[pl.ds(kbase, 512)], idxbuf.at[pl.ds(0, 512)])
        _pipe_gather_out(table, idxbuf, mc, pl.multiple_of(gid * 512, 512),
                         4, bufs, gsems)

    base = pl.multiple_of(s * REG_CHUNK, REG_CHUNK)
    pltpu.sync_copy(reg_idx.at[pl.ds(base, REG_CHUNK)], idxbuf)
    pltpu.sync_copy(reg_tok.at[pl.ds(base, REG_CHUNK)], tokbuf)
    srow = pl.multiple_of(s * QT, QT)

    for q in range(NPASS):
        lo = c * (NPASS * Q) + q * Q
        obase = pl.multiple_of(lo + s * QT, QT)

        pltpu.sync_copy(z128, buf0)
        pltpu.async_copy(buf0, acc.at[pl.ds(srow, 128)], ssemA)
        pltpu.async_copy(buf0, acc.at[pl.ds(srow + 128, 128)], ssemA)
        nmatch = _compact(idxbuf, lambda i: tokbuf[pl.ds(i * L, L)],
                          ctok, cloc, lo, lo + Q, REG_CHUNK // L, lane)
        pltpu.make_async_copy(buf0, acc.at[pl.ds(srow, 128)], ssemA).wait()
        pltpu.make_async_copy(buf0, acc.at[pl.ds(srow, 128)], ssemA).wait()
        plsc.subcore_barrier()

        _pipe_gather_scatter(table, ctok, cloc, bufs, gsems, acc,
                             nmatch, REG_CHUNK // 128)
        plsc.subcore_barrier()
        pltpu.sync_copy(acc.at[pl.ds(srow, 128)], buf0)
        pltpu.async_copy(buf0, sreg.at[pl.ds(obase, 128)], ssemA)
        pltpu.sync_copy(acc.at[pl.ds(srow + 128, 128)], buf1)
        pltpu.async_copy(buf1, sreg.at[pl.ds(obase + 128, 128)], ssemA)

        pltpu.sync_copy(z128, buf2)
        pltpu.make_async_copy(buf0, sreg.at[pl.ds(obase, 128)], ssemA).wait()
        pltpu.make_async_copy(buf0, sreg.at[pl.ds(obase, 128)], ssemA).wait()
        pltpu.sync_copy(buf2, acc.at[pl.ds(srow, 128)])
        pltpu.sync_copy(buf2, acc.at[pl.ds(srow + 128, 128)])
        pltpu.sync_copy(o128, buf2)
        plsc.subcore_barrier()

        for g in range(REG_CHUNK // 128):
            @pl.when(g * 128 < nmatch)
            def _():
                pltpu.async_copy(buf2, acc.at[cloc.at[g]], ssemA, add=True)
        for g in range(REG_CHUNK // 128):
            @pl.when(g * 128 < nmatch)
            def _():
                pltpu.make_async_copy(buf2, acc.at[cloc.at[0]], ssemA).wait()

        plsc.subcore_barrier()
        pltpu.sync_copy(acc.at[pl.ds(srow, 128)], buf0)
        pltpu.async_copy(buf0, cnt.at[pl.ds(obase, 128)], ssemA)
        pltpu.sync_copy(acc.at[pl.ds(srow + 128, 128)], buf1)
        pltpu.sync_copy(buf1, cnt.at[pl.ds(obase + 128, 128)])
        pltpu.make_async_copy(buf0, cnt.at[pl.ds(obase, 128)], ssemA).wait()


_sc1 = pl.kernel(
    _sc1_body,
    out_type=(
        jax.ShapeDtypeStruct((B, D), f32),
        jax.ShapeDtypeStruct((NM, D), f32),
        jax.ShapeDtypeStruct((NM, D), f32),
        jax.ShapeDtypeStruct((NM, D), f32),
        jax.ShapeDtypeStruct((B, D), f32),
        jax.ShapeDtypeStruct((B, D), f32),
    ),
    mesh=plsc.VectorSubcoreMesh(core_axis_name="c", subcore_axis_name="s",
                                num_cores=NC, num_subcores=NS),
    scratch_types=(
        pltpu.VMEM((REG_CHUNK,), i32),
        pltpu.VMEM((REG_CHUNK,), i32),
        pltpu.VMEM((REG_CHUNK,), i32),
        pltpu.VMEM((REG_CHUNK // 128, 128), i32),
        pltpu.VMEM((128, D), f32),
        pltpu.VMEM((128, D), f32),
        pltpu.VMEM((128, D), f32),
        pltpu.SemaphoreType.DMA,
        pltpu.SemaphoreType.DMA,
        pltpu.SemaphoreType.DMA,
        pltpu.SemaphoreType.DMA,
        pltpu.VMEM_SHARED((ACC_ROWS, D), f32),
    ),
    compiler_params=pltpu.CompilerParams(needs_layout_passes=False),
)


def _sc2_body(rm, ops1, ops2, idx_imm, idx_mem, m_out,
              idxbuf, ctok, cloc, buf0, buf1, buf2,
              gsem0, gsem1, gsem2, ssemA, acc):
    c = lax.axis_index("c")
    s = lax.axis_index("s")
    lane = jnp.arange(L, dtype=i32)
    srow = pl.multiple_of(s * QT, QT)
    base = pl.multiple_of(s * SC2_CHUNK, SC2_CHUNK)
    bufs = (buf0, buf1, buf2)
    gsems = (gsem0, gsem1, gsem2)

    for q in range(NPASS):
        lo = c * (NPASS * Q) + q * Q
        obase = pl.multiple_of(lo + s * QT, QT)
        pltpu.sync_copy(rm.at[pl.ds(obase, 128)], buf0)
        pltpu.sync_copy(rm.at[pl.ds(obase + 128, 128)], buf1)
        pltpu.sync_copy(buf0, acc.at[pl.ds(srow, 128)])
        pltpu.sync_copy(buf1, acc.at[pl.ds(srow + 128, 128)])
        plsc.subcore_barrier()

        for idx_hbm, src_hbm in ((idx_imm, ops1), (idx_mem, ops2)):
            pltpu.sync_copy(idx_hbm.at[pl.ds(base, SC2_CHUNK)], idxbuf)
            nmatch = _compact(idxbuf, lambda i: base + i * L + lane,
                              ctok, cloc, lo, lo + Q, SC2_CHUNK // L, lane)
            _pipe_gather_scatter(src_hbm, ctok, cloc, bufs, gsems,
                                 acc, nmatch, SC2_CHUNK // 128)

        plsc.subcore_barrier()
        pltpu.sync_copy(acc.at[pl.ds(srow, 128)], buf0)
        pltpu.async_copy(buf0, m_out.at[pl.ds(obase, 128)], ssemA)
        pltpu.sync_copy(acc.at[pl.ds(srow + 128, 128)], buf1)
        pltpu.sync_copy(buf1, m_out.at[pl.ds(obase + 128, 128)])
        pltpu.make_async_copy(buf0, m_out.at[pl.ds(obase, 128)], ssemA).wait()


_sc2 = pl.kernel(
    _sc2_body,
    out_type=jax.ShapeDtypeStruct((B, D), f32),
    mesh=plsc.VectorSubcoreMesh(core_axis_name="c", subcore_axis_name="s",
                                num_cores=NC, num_subcores=NS),
    scratch_types=(
        pltpu.VMEM((SC2_CHUNK,), i32),
        pltpu.VMEM((SC2_CHUNK,), i32),
        pltpu.VMEM((SC2_CHUNK // 128, 128), i32),
        pltpu.VMEM((128, D), f32),
        pltpu.VMEM((128, D), f32),
        pltpu.VMEM((128, D), f32),
        pltpu.SemaphoreType.DMA,
        pltpu.SemaphoreType.DMA,
        pltpu.SemaphoreType.DMA,
        pltpu.SemaphoreType.DMA,
        pltpu.VMEM_SHARED((ACC_ROWS, D), f32),
    ),
    compiler_params=pltpu.CompilerParams(needs_layout_passes=False),
)


RB = 1024


def _tcb_body(imm_ref, disp_ref, mc0_ref, mc1_ref, mc2_ref,
              sreg_ref, cnt_ref, mnem_ref,
              iW1, ib1, iW2, ib2, mW1, mb1, mW2, mb2,
              oW0, ob0, oW1, ob1, oW2, ob2,
              ops1_ref, ops2_ref, r_ref):
    w1i = iW1[...]
    b1i = ib1[...][None, :]
    w2i = iW2[...]
    b2i = ib2[...][None, :]

    x = _imm_pre(imm_ref[...])
    h = _leaky(x * w1i + b1i)
    imm_e = _leaky(jnp.dot(h, w2i, preferred_element_type=f32) + b2i)
    ops1_ref[...] = jnp.dot(imm_e, oW1[...], preferred_element_type=f32) \
        + ob1[...][None, :]

    dx = _imm_pre(disp_ref[...])
    hd = _leaky(dx * w1i + b1i)
    disp_e = _leaky(jnp.dot(hd, w2i, preferred_element_type=f32) + b2i)

    w1m = mW1[...]
    m1 = (jnp.dot(mc0_ref[...], w1m[:D], preferred_element_type=f32)
          + jnp.dot(mc1_ref[...], w1m[D:2 * D], preferred_element_type=f32)
          + jnp.dot(mc2_ref[...], w1m[2 * D:3 * D], preferred_element_type=f32)
          + jnp.dot(disp_e, w1m[3 * D:], preferred_element_type=f32)
          + mb1[...][None, :])
    mem_e = _leaky(jnp.dot(_leaky(m1), mW2[...], preferred_element_type=f32)
                   + mb2[...][None, :])
    ops2_ref[...] = jnp.dot(mem_e, oW2[...], preferred_element_type=f32) \
        + ob2[...][None, :]

    r_ref[...] = (mnem_ref[...]
                  + jnp.dot(sreg_ref[...], oW0[...], preferred_element_type=f32)
                  + cnt_ref[:, 0:1] * ob0[...][None, :])


def _row_spec(cols):
    return pl.BlockSpec((RB, cols), lambda i: (i, 0))


def _full2(shape):
    return pl.BlockSpec(shape, lambda i: (0, 0))


def _full1(n):
    return pl.BlockSpec((n,), lambda i: (0,))


_tcb = pl.pallas_call(
    _tcb_body,
    grid=(B // RB,),
    in_specs=[
        _row_spec(1), _row_spec(1), _row_spec(D), _row_spec(D), _row_spec(D),
        _row_spec(D), _row_spec(D), _row_spec(D),
        _full2((1, D)), _full1(D), _full2((D, D)), _full1(D),
        _full2((4 * D, D)), _full1(D), _full2((D, D)), _full1(D),
        _full2((D, D)), _full1(D), _full2((D, D)), _full1(D),
        _full2((D, D)), _full1(D),
    ],
    out_specs=[_row_spec(D)] * 3,
    out_shape=[jax.ShapeDtypeStruct((B, D), f32)] * 3,
)


def _tcd_body(m_ref, aggW_ref, aggb_ref, out_ref):
    out_ref[...] = _leaky(jnp.dot(_leaky(m_ref[...]), aggW_ref[...],
                                  preferred_element_type=f32)
                          + aggb_ref[...][None, :])


_tcd = pl.pallas_call(
    _tcd_body,
    grid=(B // RB,),
    in_specs=[_row_spec(D), _full2((D, D)), _full1(D)],
    out_specs=_row_spec(D),
    out_shape=jax.ShapeDtypeStruct((B, D), f32),
)


def kernel(mnemic, reg_tokens, imm_vals, mem_tokens, mem_disp,
           ins_idx_reg, ins_idx_imm, ins_idx_mem,
           table, imm_W1, imm_b1, imm_W2, imm_b2,
           mem_W1, mem_b1, mem_W2, mem_b2,
           opW0, opb0, opW1, opb1, opW2, opb2, aggW, aggb):
    mnemic = mnemic.astype(i32)
    reg_tokens = reg_tokens.astype(i32)
    mem_flat = mem_tokens.astype(i32).T.reshape(-1)
    iir = ins_idx_reg.astype(i32)
    iii = ins_idx_imm.astype(i32)
    iim = ins_idx_mem.astype(i32)
    z128 = jnp.zeros((128, D), f32)
    o128 = jnp.ones((128, D), f32)

    mnem_g, mc0, mc1, mc2, sreg, cnt = _sc1(table, mnemic, reg_tokens, iir,
                                            mem_flat, z128, o128)
    ops1, ops2, r = _tcb(imm_vals, mem_disp.reshape(NM, 1),
                         mc0, mc1, mc2, sreg, cnt, mnem_g,
                         imm_W1, imm_b1, imm_W2, imm_b2,
                         mem_W1, mem_b1, mem_W2, mem_b2,
                         opW0, opb0, opW1, opb1, opW2, opb2)
    m = _sc2(r, ops1, ops2, iii, iim)
    return _tcd(m, aggW, aggb)
